# baseline (speedup 1.0000x reference)
import functools

import jax
import jax.numpy as jnp
from jax import lax
from jax.experimental import pallas as pl
from jax.experimental.pallas import tpu as pltpu
from jax.experimental.pallas import tpu_sc as plsc

_L = 16


def _make_sc_scores(B, NEG, V, D):
    info = plsc.get_sparse_core_info()
    NC, NS = info.num_cores, info.num_subcores
    NW = NC * NS
    b_per_w = B // NW
    CHUNK = 64
    n_chunks = b_per_w // CHUNK
    D2 = 2 * D
    mesh = plsc.VectorSubcoreMesh(core_axis_name="c", subcore_axis_name="s")

    @functools.partial(
        pl.kernel,
        out_type=jax.ShapeDtypeStruct((8 * B,), jnp.float32),
        mesh=mesh,
        compiler_params=pltpu.CompilerParams(
            use_tc_tiling_on_sc=True, needs_layout_passes=False),
        scratch_types=[
            pltpu.VMEM((CHUNK,), jnp.int32),
            pltpu.VMEM((CHUNK,), jnp.int32),
            pltpu.VMEM((NEG * CHUNK,), jnp.int32),
            pltpu.VMEM((CHUNK, D2), jnp.float32),
            pltpu.VMEM((CHUNK, D2), jnp.float32),
            pltpu.VMEM((NEG, CHUNK, D2), jnp.float32),
            pltpu.VMEM((8 * CHUNK,), jnp.float32),
            pltpu.SemaphoreType.DMA,
        ],
    )
    def sc_scores(cen_hbm, pos_hbm, negt_hbm, itab_hbm, otab_hbm, out_hbm,
                  idx_c, idx_p, idx_n, cen_v, pos_v, neg_v, sc_v, sem):
        wid = lax.axis_index("s") * NC + lax.axis_index("c")

        def chunk_body(ci, carry):
            base = wid * b_per_w + ci * CHUNK
            pltpu.sync_copy(cen_hbm.at[pl.ds(base, CHUNK)], idx_c)
            pltpu.sync_copy(pos_hbm.at[pl.ds(base, CHUNK)], idx_p)
            for k in range(NEG):
                pltpu.sync_copy(negt_hbm.at[pl.ds(k * B + base, CHUNK)],
                                idx_n.at[pl.ds(k * CHUNK, CHUNK)])
            h_c = pltpu.async_copy(itab_hbm.at[idx_c], cen_v, sem)
            h_p = pltpu.async_copy(otab_hbm.at[idx_p], pos_v, sem)
            h_n = [pltpu.async_copy(
                       otab_hbm.at[idx_n.at[pl.ds(k * CHUNK, CHUNK)]],
                       neg_v.at[k], sem)
                   for k in range(NEG)]
            h_c.wait()
            h_p.wait()
            for h in h_n:
                h.wait()

            def group_body(g, carry2):
                rows = g * _L + lax.broadcasted_iota(jnp.int32, (_L,), 0)

                def d_body(dd, accs):
                    acc_p, acc_n = accs
                    dsplat = jnp.full((_L,), dd, jnp.int32)
                    c = plsc.load_gather(cen_v, [rows, dsplat])
                    p = plsc.load_gather(pos_v, [rows, dsplat])
                    acc_p = acc_p + c * p
                    new_n = tuple(
                        acc_n[k] + c * plsc.load_gather(
                            neg_v,
                            [jnp.full((_L,), k, jnp.int32), rows, dsplat])
                        for k in range(NEG))
                    return acc_p, new_n

                z = jnp.zeros((_L,), jnp.float32)
                acc_p, acc_n = lax.fori_loop(0, D, d_body, (z, (z,) * NEG))
                sc_v[pl.ds(0 * CHUNK + g * _L, _L)] = acc_p
                for k in range(NEG):
                    sc_v[pl.ds((1 + k) * CHUNK + g * _L, _L)] = acc_n[k]
                sc_v[pl.ds(6 * CHUNK + g * _L, _L)] = z
                sc_v[pl.ds(7 * CHUNK + g * _L, _L)] = z
                return carry2

            lax.fori_loop(0, CHUNK // _L, group_body, 0)
            for r in range(8):
                pltpu.sync_copy(sc_v.at[pl.ds(r * CHUNK, CHUNK)],
                                out_hbm.at[pl.ds(r * B + base, CHUNK)])
            return carry

        lax.fori_loop(0, n_chunks, chunk_body, 0)

    return sc_scores


def _relayout_body(a_ref, b_ref, oa_ref, ob_ref):
    ta = jnp.transpose(a_ref[...], (1, 0))
    tb = jnp.transpose(b_ref[...], (1, 0))
    oa_ref[...] = jnp.concatenate([ta, ta], axis=1)
    ob_ref[...] = jnp.concatenate([tb, tb], axis=1)


def _relayout_pair(itab, otab, V, D):
    itab_t = jnp.transpose(itab)
    otab_t = jnp.transpose(otab)
    cols = 1024
    grid = (V + cols - 1) // cols
    ispec = pl.BlockSpec((D, cols), lambda g: (0, g))
    ospec = pl.BlockSpec((cols, 2 * D), lambda g: (g, 0))
    oshape = jax.ShapeDtypeStruct((V, 2 * D), jnp.float32)
    return pl.pallas_call(
        _relayout_body,
        grid=(grid,),
        in_specs=[ispec, ispec],
        out_specs=[ospec, ospec],
        out_shape=[oshape, oshape],
    )(itab_t, otab_t)


def _loss_body(s_ref, o_ref):
    s = s_ref[...]
    rows = lax.broadcasted_iota(jnp.int32, s.shape, 0)
    x = jnp.where(rows == 0, s, -s)
    l = jnp.log(jax.nn.sigmoid(x) + 1e-10)
    l = jnp.where(rows < 6, l, 0.0)
    o_ref[0, 0] = -jnp.sum(l) / s.shape[1]


def kernel(center_words, pos_words, neg_words, input_table, output_table):
    B, = center_words.shape
    NEG = neg_words.shape[1]
    V, D = input_table.shape
    neg_t = jnp.reshape(jnp.transpose(neg_words), (-1,))
    itab2, otab2 = _relayout_pair(input_table, output_table, V, D)
    sc_scores = _make_sc_scores(B, NEG, V, D)
    scores = jnp.reshape(
        sc_scores(center_words, pos_words, neg_t, itab2, otab2), (8, B))
    loss = pl.pallas_call(
        _loss_body,
        out_shape=jax.ShapeDtypeStruct((1, 1), jnp.float32),
        out_specs=pl.BlockSpec(memory_space=pltpu.SMEM),
    )(scores)
    return jnp.reshape(loss, ())


# speedup vs baseline: 1.4124x; 1.4124x over previous
import functools

import jax
import jax.numpy as jnp
from jax import lax
from jax.experimental import pallas as pl
from jax.experimental.pallas import tpu as pltpu
from jax.experimental.pallas import tpu_sc as plsc

_L = 16


def _make_sc_scores(B, NEG, V, D):
    info = plsc.get_sparse_core_info()
    NC, NS = info.num_cores, info.num_subcores
    NW = NC * NS
    b_per_w = B // NW
    CHUNK = 64
    n_chunks = b_per_w // CHUNK
    D2 = 2 * D
    mesh = plsc.VectorSubcoreMesh(core_axis_name="c", subcore_axis_name="s")

    @functools.partial(
        pl.kernel,
        out_type=jax.ShapeDtypeStruct((8 * B,), jnp.float32),
        mesh=mesh,
        compiler_params=pltpu.CompilerParams(
            use_tc_tiling_on_sc=True, needs_layout_passes=False),
        scratch_types=[
            pltpu.VMEM((CHUNK,), jnp.int32),
            pltpu.VMEM((CHUNK,), jnp.int32),
            pltpu.VMEM((NEG * CHUNK,), jnp.int32),
            pltpu.VMEM((CHUNK, D2), jnp.float32),
            pltpu.VMEM((CHUNK, D2), jnp.float32),
            pltpu.VMEM((NEG, CHUNK, D2), jnp.float32),
            pltpu.VMEM((8 * CHUNK,), jnp.float32),
            pltpu.SemaphoreType.DMA,
        ],
    )
    def sc_scores(cen_hbm, pos_hbm, negt_hbm, itab_hbm, otab_hbm, out_hbm,
                  idx_c, idx_p, idx_n, cen_v, pos_v, neg_v, sc_v, sem):
        wid = lax.axis_index("s") * NC + lax.axis_index("c")

        def chunk_body(ci, carry):
            base = wid * b_per_w + ci * CHUNK
            pltpu.sync_copy(cen_hbm.at[pl.ds(base, CHUNK)], idx_c)
            pltpu.sync_copy(pos_hbm.at[pl.ds(base, CHUNK)], idx_p)
            for k in range(NEG):
                pltpu.sync_copy(negt_hbm.at[pl.ds(k * B + base, CHUNK)],
                                idx_n.at[pl.ds(k * CHUNK, CHUNK)])
            h_c = pltpu.async_copy(itab_hbm.at[idx_c], cen_v, sem)
            h_p = pltpu.async_copy(otab_hbm.at[idx_p], pos_v, sem)
            h_n = [pltpu.async_copy(
                       otab_hbm.at[idx_n.at[pl.ds(k * CHUNK, CHUNK)]],
                       neg_v.at[k], sem)
                   for k in range(NEG)]
            h_c.wait()
            h_p.wait()
            for h in h_n:
                h.wait()

            def group_body(g, carry2):
                rows = g * _L + lax.broadcasted_iota(jnp.int32, (_L,), 0)

                def d_body(dd, accs):
                    acc_p, acc_n = accs
                    dsplat = jnp.full((_L,), dd, jnp.int32)
                    c = plsc.load_gather(cen_v, [rows, dsplat])
                    p = plsc.load_gather(pos_v, [rows, dsplat])
                    acc_p = acc_p + c * p
                    new_n = tuple(
                        acc_n[k] + c * plsc.load_gather(
                            neg_v,
                            [jnp.full((_L,), k, jnp.int32), rows, dsplat])
                        for k in range(NEG))
                    return acc_p, new_n

                z = jnp.zeros((_L,), jnp.float32)
                acc_p, acc_n = lax.fori_loop(0, D, d_body, (z, (z,) * NEG))
                sc_v[pl.ds(0 * CHUNK + g * _L, _L)] = acc_p
                for k in range(NEG):
                    sc_v[pl.ds((1 + k) * CHUNK + g * _L, _L)] = acc_n[k]
                sc_v[pl.ds(6 * CHUNK + g * _L, _L)] = z
                sc_v[pl.ds(7 * CHUNK + g * _L, _L)] = z
                return carry2

            lax.fori_loop(0, CHUNK // _L, group_body, 0)
            for r in range(8):
                pltpu.sync_copy(sc_v.at[pl.ds(r * CHUNK, CHUNK)],
                                out_hbm.at[pl.ds(r * B + base, CHUNK)])
            return carry

        lax.fori_loop(0, n_chunks, chunk_body, 0)

    return sc_scores


def _relayout_body(a_ref, b_ref, oa_ref, ob_ref):
    oa_ref[:, pl.ds(0, 64)] = jnp.transpose(a_ref[...], (1, 0))
    ob_ref[:, pl.ds(0, 64)] = jnp.transpose(b_ref[...], (1, 0))


def _relayout_pair(itab, otab, V, D):
    itab_t = jnp.transpose(itab)
    otab_t = jnp.transpose(otab)
    cols = 2048
    grid = (V + cols - 1) // cols
    ispec = pl.BlockSpec((D, cols), lambda g: (0, g))
    ospec = pl.BlockSpec((cols, 2 * D), lambda g: (g, 0))
    oshape = jax.ShapeDtypeStruct((V, 2 * D), jnp.float32)
    return pl.pallas_call(
        _relayout_body,
        grid=(grid,),
        in_specs=[ispec, ispec],
        out_specs=[ospec, ospec],
        out_shape=[oshape, oshape],
    )(itab_t, otab_t)


def _loss_body(s_ref, o_ref):
    s = s_ref[...]
    rows = lax.broadcasted_iota(jnp.int32, s.shape, 0)
    x = jnp.where(rows == 0, s, -s)
    l = jnp.log(jax.nn.sigmoid(x) + 1e-10)
    l = jnp.where(rows < 6, l, 0.0)
    o_ref[0, 0] = -jnp.sum(l) / s.shape[1]


def kernel(center_words, pos_words, neg_words, input_table, output_table):
    B, = center_words.shape
    NEG = neg_words.shape[1]
    V, D = input_table.shape
    neg_t = jnp.reshape(jnp.transpose(neg_words), (-1,))
    itab2, otab2 = _relayout_pair(input_table, output_table, V, D)
    sc_scores = _make_sc_scores(B, NEG, V, D)
    scores = jnp.reshape(
        sc_scores(center_words, pos_words, neg_t, itab2, otab2), (8, B))
    loss = pl.pallas_call(
        _loss_body,
        out_shape=jax.ShapeDtypeStruct((1, 1), jnp.float32),
        out_specs=pl.BlockSpec(memory_space=pltpu.SMEM),
    )(scores)
    return jnp.reshape(loss, ())


# speedup vs baseline: 1.6943x; 1.1996x over previous
import functools

import jax
import jax.numpy as jnp
from jax import lax
from jax.experimental import pallas as pl
from jax.experimental.pallas import tpu as pltpu
from jax.experimental.pallas import tpu_sc as plsc

_L = 16


def _make_sc_scores(B, NEG, V, D):
    info = plsc.get_sparse_core_info()
    NC, NS = info.num_cores, info.num_subcores
    NW = NC * NS
    b_per_w = B // NW
    CHUNK = 64
    n_chunks = b_per_w // CHUNK
    D2 = 2 * D
    mesh = plsc.VectorSubcoreMesh(core_axis_name="c", subcore_axis_name="s")

    @functools.partial(
        pl.kernel,
        out_type=jax.ShapeDtypeStruct((8 * B,), jnp.float32),
        mesh=mesh,
        compiler_params=pltpu.CompilerParams(
            use_tc_tiling_on_sc=True, needs_layout_passes=False),
        scratch_types=[
            pltpu.VMEM((CHUNK,), jnp.int32),
            pltpu.VMEM((CHUNK,), jnp.int32),
            pltpu.VMEM((NEG * CHUNK,), jnp.int32),
            pltpu.VMEM((CHUNK, D2), jnp.float32),
            pltpu.VMEM((CHUNK, D2), jnp.float32),
            pltpu.VMEM((NEG, CHUNK, D2), jnp.float32),
            pltpu.VMEM((8 * CHUNK,), jnp.float32),
            pltpu.SemaphoreType.DMA,
        ],
    )
    def sc_scores(cen_hbm, pos_hbm, negt_hbm, itab_hbm, otab_hbm, out_hbm,
                  idx_c, idx_p, idx_n, cen_v, pos_v, neg_v, sc_v, sem):
        wid = lax.axis_index("s") * NC + lax.axis_index("c")

        def chunk_body(ci, carry):
            base = wid * b_per_w + ci * CHUNK
            pltpu.sync_copy(cen_hbm.at[pl.ds(base, CHUNK)], idx_c)
            pltpu.sync_copy(pos_hbm.at[pl.ds(base, CHUNK)], idx_p)
            for k in range(NEG):
                pltpu.sync_copy(negt_hbm.at[pl.ds(k * B + base, CHUNK)],
                                idx_n.at[pl.ds(k * CHUNK, CHUNK)])
            h_c = pltpu.async_copy(itab_hbm.at[idx_c], cen_v, sem)
            h_p = pltpu.async_copy(otab_hbm.at[idx_p], pos_v, sem)
            h_n = [pltpu.async_copy(
                       otab_hbm.at[idx_n.at[pl.ds(k * CHUNK, CHUNK)]],
                       neg_v.at[k], sem)
                   for k in range(NEG)]
            h_c.wait()
            h_p.wait()
            for h in h_n:
                h.wait()

            def group_body(g, carry2):
                rows = g * _L + lax.broadcasted_iota(jnp.int32, (_L,), 0)

                def d_body(dd, accs):
                    acc_p, acc_n = accs
                    dsplat = jnp.full((_L,), dd, jnp.int32)
                    c = plsc.load_gather(cen_v, [rows, dsplat])
                    p = plsc.load_gather(pos_v, [rows, dsplat])
                    acc_p = acc_p + c * p
                    new_n = tuple(
                        acc_n[k] + c * plsc.load_gather(
                            neg_v,
                            [jnp.full((_L,), k, jnp.int32), rows, dsplat])
                        for k in range(NEG))
                    return acc_p, new_n

                z = jnp.zeros((_L,), jnp.float32)
                acc_p, acc_n = lax.fori_loop(0, D, d_body, (z, (z,) * NEG))
                sc_v[pl.ds(0 * CHUNK + g * _L, _L)] = acc_p
                for k in range(NEG):
                    sc_v[pl.ds((1 + k) * CHUNK + g * _L, _L)] = acc_n[k]
                sc_v[pl.ds(6 * CHUNK + g * _L, _L)] = z
                sc_v[pl.ds(7 * CHUNK + g * _L, _L)] = z
                return carry2

            lax.fori_loop(0, CHUNK // _L, group_body, 0)
            for r in range(8):
                pltpu.sync_copy(sc_v.at[pl.ds(r * CHUNK, CHUNK)],
                                out_hbm.at[pl.ds(r * B + base, CHUNK)])
            return carry

        lax.fori_loop(0, n_chunks, chunk_body, 0)

    return sc_scores


def _relayout_body(a_ref, b_ref, oa_ref, ob_ref):
    oa_ref[:, pl.ds(0, 64)] = jnp.transpose(a_ref[...], (1, 0))
    ob_ref[:, pl.ds(0, 64)] = jnp.transpose(b_ref[...], (1, 0))


def _relayout_pair(itab, otab, V, D):
    itab_t = jnp.transpose(itab)
    otab_t = jnp.transpose(otab)
    cols = 4096
    grid = (V + cols - 1) // cols
    ispec = pl.BlockSpec((D, cols), lambda g: (0, g))
    ospec = pl.BlockSpec((cols, 2 * D), lambda g: (g, 0))
    oshape = jax.ShapeDtypeStruct((V, 2 * D), jnp.float32)
    return pl.pallas_call(
        _relayout_body,
        grid=(grid,),
        in_specs=[ispec, ispec],
        out_specs=[ospec, ospec],
        out_shape=[oshape, oshape],
    )(itab_t, otab_t)


def _loss_body(s_ref, o_ref):
    s = s_ref[...]
    rows = lax.broadcasted_iota(jnp.int32, s.shape, 0)
    x = jnp.where(rows == 0, s, -s)
    l = jnp.log(jax.nn.sigmoid(x) + 1e-10)
    l = jnp.where(rows < 6, l, 0.0)
    o_ref[0, 0] = -jnp.sum(l) / s.shape[1]


def kernel(center_words, pos_words, neg_words, input_table, output_table):
    B, = center_words.shape
    NEG = neg_words.shape[1]
    V, D = input_table.shape
    neg_t = jnp.reshape(jnp.transpose(neg_words), (-1,))
    itab2, otab2 = _relayout_pair(input_table, output_table, V, D)
    sc_scores = _make_sc_scores(B, NEG, V, D)
    scores = jnp.reshape(
        sc_scores(center_words, pos_words, neg_t, itab2, otab2), (8, B))
    loss = pl.pallas_call(
        _loss_body,
        out_shape=jax.ShapeDtypeStruct((1, 1), jnp.float32),
        out_specs=pl.BlockSpec(memory_space=pltpu.SMEM),
    )(scores)
    return jnp.reshape(loss, ())


# speedup vs baseline: 1.7965x; 1.0604x over previous
import functools

import jax
import jax.numpy as jnp
from jax import lax
from jax.experimental import pallas as pl
from jax.experimental.pallas import tpu as pltpu
from jax.experimental.pallas import tpu_sc as plsc

_L = 16


def _make_sc_scores(B, NEG, V, D):
    info = plsc.get_sparse_core_info()
    NC, NS = info.num_cores, info.num_subcores
    NW = NC * NS
    b_per_w = B // NW
    CHUNK = 64
    n_chunks = b_per_w // CHUNK
    D2 = 2 * D
    mesh = plsc.VectorSubcoreMesh(core_axis_name="c", subcore_axis_name="s")

    @functools.partial(
        pl.kernel,
        out_type=jax.ShapeDtypeStruct((8 * B,), jnp.float32),
        mesh=mesh,
        compiler_params=pltpu.CompilerParams(
            use_tc_tiling_on_sc=True, needs_layout_passes=False),
        scratch_types=[
            pltpu.VMEM((CHUNK,), jnp.int32),
            pltpu.VMEM((CHUNK,), jnp.int32),
            pltpu.VMEM((NEG * CHUNK,), jnp.int32),
            pltpu.VMEM((CHUNK, D2), jnp.float32),
            pltpu.VMEM((CHUNK, D2), jnp.float32),
            pltpu.VMEM((NEG, CHUNK, D2), jnp.float32),
            pltpu.VMEM((8 * CHUNK,), jnp.float32),
            pltpu.SemaphoreType.DMA,
        ],
    )
    def sc_scores(cen_hbm, pos_hbm, negt_hbm, itab_hbm, otab_hbm, out_hbm,
                  idx_c, idx_p, idx_n, cen_v, pos_v, neg_v, sc_v, sem):
        wid = lax.axis_index("s") * NC + lax.axis_index("c")

        def chunk_body(ci, carry):
            base = wid * b_per_w + ci * CHUNK
            pltpu.sync_copy(cen_hbm.at[pl.ds(base, CHUNK)], idx_c)
            pltpu.sync_copy(pos_hbm.at[pl.ds(base, CHUNK)], idx_p)
            for k in range(NEG):
                pltpu.sync_copy(negt_hbm.at[pl.ds(k * B + base, CHUNK)],
                                idx_n.at[pl.ds(k * CHUNK, CHUNK)])
            h_c = pltpu.async_copy(itab_hbm.at[idx_c], cen_v, sem)
            h_p = pltpu.async_copy(otab_hbm.at[idx_p], pos_v, sem)
            h_n = [pltpu.async_copy(
                       otab_hbm.at[idx_n.at[pl.ds(k * CHUNK, CHUNK)]],
                       neg_v.at[k], sem)
                   for k in range(NEG)]
            h_c.wait()
            h_p.wait()
            for h in h_n:
                h.wait()

            def group_body(g, carry2):
                rows = g * _L + lax.broadcasted_iota(jnp.int32, (_L,), 0)

                def d_body(dd, accs):
                    acc_p, acc_n = accs
                    dsplat = jnp.full((_L,), dd, jnp.int32)
                    c = plsc.load_gather(cen_v, [rows, dsplat])
                    p = plsc.load_gather(pos_v, [rows, dsplat])
                    acc_p = acc_p + c * p
                    new_n = tuple(
                        acc_n[k] + c * plsc.load_gather(
                            neg_v,
                            [jnp.full((_L,), k, jnp.int32), rows, dsplat])
                        for k in range(NEG))
                    return acc_p, new_n

                z = jnp.zeros((_L,), jnp.float32)
                acc_p, acc_n = lax.fori_loop(0, D, d_body, (z, (z,) * NEG))
                sc_v[pl.ds(0 * CHUNK + g * _L, _L)] = acc_p
                for k in range(NEG):
                    sc_v[pl.ds((1 + k) * CHUNK + g * _L, _L)] = acc_n[k]
                sc_v[pl.ds(6 * CHUNK + g * _L, _L)] = z
                sc_v[pl.ds(7 * CHUNK + g * _L, _L)] = z
                return carry2

            lax.fori_loop(0, CHUNK // _L, group_body, 0)
            for r in range(8):
                pltpu.sync_copy(sc_v.at[pl.ds(r * CHUNK, CHUNK)],
                                out_hbm.at[pl.ds(r * B + base, CHUNK)])
            return carry

        lax.fori_loop(0, n_chunks, chunk_body, 0)

    return sc_scores


def _relayout_body(a_ref, b_ref, oa_ref, ob_ref):
    oa_ref[:, pl.ds(0, 64)] = jnp.transpose(a_ref[...], (1, 0))
    ob_ref[:, pl.ds(0, 64)] = jnp.transpose(b_ref[...], (1, 0))


def _relayout_pair(itab, otab, V, D):
    itab_t = jnp.transpose(itab)
    otab_t = jnp.transpose(otab)
    cols = 8192
    grid = (V + cols - 1) // cols
    ispec = pl.BlockSpec((D, cols), lambda g: (0, g))
    ospec = pl.BlockSpec((cols, 2 * D), lambda g: (g, 0))
    oshape = jax.ShapeDtypeStruct((V, 2 * D), jnp.float32)
    return pl.pallas_call(
        _relayout_body,
        grid=(grid,),
        in_specs=[ispec, ispec],
        out_specs=[ospec, ospec],
        out_shape=[oshape, oshape],
    )(itab_t, otab_t)


def _loss_body(s_ref, o_ref):
    s = s_ref[...]
    rows = lax.broadcasted_iota(jnp.int32, s.shape, 0)
    x = jnp.where(rows == 0, s, -s)
    l = jnp.log(jax.nn.sigmoid(x) + 1e-10)
    l = jnp.where(rows < 6, l, 0.0)
    o_ref[0, 0] = -jnp.sum(l) / s.shape[1]


def kernel(center_words, pos_words, neg_words, input_table, output_table):
    B, = center_words.shape
    NEG = neg_words.shape[1]
    V, D = input_table.shape
    neg_t = jnp.reshape(jnp.transpose(neg_words), (-1,))
    itab2, otab2 = _relayout_pair(input_table, output_table, V, D)
    sc_scores = _make_sc_scores(B, NEG, V, D)
    scores = jnp.reshape(
        sc_scores(center_words, pos_words, neg_t, itab2, otab2), (8, B))
    loss = pl.pallas_call(
        _loss_body,
        out_shape=jax.ShapeDtypeStruct((1, 1), jnp.float32),
        out_specs=pl.BlockSpec(memory_space=pltpu.SMEM),
    )(scores)
    return jnp.reshape(loss, ())


# speedup vs baseline: 1.8318x; 1.0196x over previous
import functools

import jax
import jax.numpy as jnp
from jax import lax
from jax.experimental import pallas as pl
from jax.experimental.pallas import tpu as pltpu
from jax.experimental.pallas import tpu_sc as plsc

_L = 16


def _make_sc_scores(B, NEG, V, D):
    info = plsc.get_sparse_core_info()
    NC, NS = info.num_cores, info.num_subcores
    NW = NC * NS
    b_per_w = B // NW
    CHUNK = 64
    n_chunks = b_per_w // CHUNK
    D2 = 2 * D
    mesh = plsc.VectorSubcoreMesh(core_axis_name="c", subcore_axis_name="s")

    @functools.partial(
        pl.kernel,
        out_type=jax.ShapeDtypeStruct((8 * B,), jnp.float32),
        mesh=mesh,
        compiler_params=pltpu.CompilerParams(
            use_tc_tiling_on_sc=True, needs_layout_passes=False),
        scratch_types=[
            pltpu.VMEM((CHUNK,), jnp.int32),
            pltpu.VMEM((CHUNK,), jnp.int32),
            pltpu.VMEM((NEG * CHUNK,), jnp.int32),
            pltpu.VMEM((CHUNK, D2), jnp.float32),
            pltpu.VMEM((CHUNK, D2), jnp.float32),
            pltpu.VMEM((NEG, CHUNK, D2), jnp.float32),
            pltpu.VMEM((8 * CHUNK,), jnp.float32),
            pltpu.SemaphoreType.DMA,
        ],
    )
    def sc_scores(cen_hbm, pos_hbm, negt_hbm, itab_hbm, otab_hbm, out_hbm,
                  idx_c, idx_p, idx_n, cen_v, pos_v, neg_v, sc_v, sem):
        wid = lax.axis_index("s") * NC + lax.axis_index("c")

        def chunk_body(ci, carry):
            base = wid * b_per_w + ci * CHUNK
            pltpu.sync_copy(cen_hbm.at[pl.ds(base, CHUNK)], idx_c)
            pltpu.sync_copy(pos_hbm.at[pl.ds(base, CHUNK)], idx_p)
            for k in range(NEG):
                pltpu.sync_copy(negt_hbm.at[pl.ds(k * B + base, CHUNK)],
                                idx_n.at[pl.ds(k * CHUNK, CHUNK)])
            h_c = pltpu.async_copy(itab_hbm.at[idx_c], cen_v, sem)
            h_p = pltpu.async_copy(otab_hbm.at[idx_p], pos_v, sem)
            h_n = [pltpu.async_copy(
                       otab_hbm.at[idx_n.at[pl.ds(k * CHUNK, CHUNK)]],
                       neg_v.at[k], sem)
                   for k in range(NEG)]
            h_c.wait()
            h_p.wait()
            for h in h_n:
                h.wait()

            def group_body(g, carry2):
                rows = g * _L + lax.broadcasted_iota(jnp.int32, (_L,), 0)

                def d_body(dd, accs):
                    acc_p, acc_n = accs
                    dsplat = jnp.full((_L,), dd, jnp.int32)
                    c = plsc.load_gather(cen_v, [rows, dsplat])
                    p = plsc.load_gather(pos_v, [rows, dsplat])
                    acc_p = acc_p + c * p
                    new_n = tuple(
                        acc_n[k] + c * plsc.load_gather(
                            neg_v,
                            [jnp.full((_L,), k, jnp.int32), rows, dsplat])
                        for k in range(NEG))
                    return acc_p, new_n

                z = jnp.zeros((_L,), jnp.float32)
                acc_p, acc_n = lax.fori_loop(0, D, d_body, (z, (z,) * NEG))
                sc_v[pl.ds(0 * CHUNK + g * _L, _L)] = acc_p
                for k in range(NEG):
                    sc_v[pl.ds((1 + k) * CHUNK + g * _L, _L)] = acc_n[k]
                sc_v[pl.ds(6 * CHUNK + g * _L, _L)] = z
                sc_v[pl.ds(7 * CHUNK + g * _L, _L)] = z
                return carry2

            lax.fori_loop(0, CHUNK // _L, group_body, 0)
            for r in range(8):
                pltpu.sync_copy(sc_v.at[pl.ds(r * CHUNK, CHUNK)],
                                out_hbm.at[pl.ds(r * B + base, CHUNK)])
            return carry

        lax.fori_loop(0, n_chunks, chunk_body, 0)

    return sc_scores


def _relayout_body(a_ref, b_ref, oa_ref, ob_ref):
    oa_ref[:, pl.ds(0, 64)] = jnp.transpose(a_ref[...], (1, 0))
    ob_ref[:, pl.ds(0, 64)] = jnp.transpose(b_ref[...], (1, 0))


def _relayout_pair(itab, otab, V, D):
    itab_t = jnp.transpose(itab)
    otab_t = jnp.transpose(otab)
    cols = 16384
    grid = (V + cols - 1) // cols
    ispec = pl.BlockSpec((D, cols), lambda g: (0, g))
    ospec = pl.BlockSpec((cols, 2 * D), lambda g: (g, 0))
    oshape = jax.ShapeDtypeStruct((V, 2 * D), jnp.float32)
    return pl.pallas_call(
        _relayout_body,
        grid=(grid,),
        in_specs=[ispec, ispec],
        out_specs=[ospec, ospec],
        out_shape=[oshape, oshape],
    )(itab_t, otab_t)


def _loss_body(s_ref, o_ref):
    s = s_ref[...]
    rows = lax.broadcasted_iota(jnp.int32, s.shape, 0)
    x = jnp.where(rows == 0, s, -s)
    l = jnp.log(jax.nn.sigmoid(x) + 1e-10)
    l = jnp.where(rows < 6, l, 0.0)
    o_ref[0, 0] = -jnp.sum(l) / s.shape[1]


def kernel(center_words, pos_words, neg_words, input_table, output_table):
    B, = center_words.shape
    NEG = neg_words.shape[1]
    V, D = input_table.shape
    neg_t = jnp.reshape(jnp.transpose(neg_words), (-1,))
    itab2, otab2 = _relayout_pair(input_table, output_table, V, D)
    sc_scores = _make_sc_scores(B, NEG, V, D)
    scores = jnp.reshape(
        sc_scores(center_words, pos_words, neg_t, itab2, otab2), (8, B))
    loss = pl.pallas_call(
        _loss_body,
        out_shape=jax.ShapeDtypeStruct((1, 1), jnp.float32),
        out_specs=pl.BlockSpec(memory_space=pltpu.SMEM),
    )(scores)
    return jnp.reshape(loss, ())


# speedup vs baseline: 2.0043x; 1.0942x over previous
import functools

import jax
import jax.numpy as jnp
from jax import lax
from jax.experimental import pallas as pl
from jax.experimental.pallas import tpu as pltpu
from jax.experimental.pallas import tpu_sc as plsc

_L = 16


def _make_sc_scores(B, NEG, V, D):
    info = plsc.get_sparse_core_info()
    NC, NS = info.num_cores, info.num_subcores
    NW = NC * NS
    b_per_w = B // NW
    CHUNK = 64
    n_chunks = b_per_w // CHUNK
    D2 = 2 * D
    mesh = plsc.VectorSubcoreMesh(core_axis_name="c", subcore_axis_name="s")

    @functools.partial(
        pl.kernel,
        out_type=jax.ShapeDtypeStruct((8 * B,), jnp.float32),
        mesh=mesh,
        compiler_params=pltpu.CompilerParams(
            use_tc_tiling_on_sc=True, needs_layout_passes=False),
        scratch_types=[
            pltpu.VMEM((CHUNK,), jnp.int32),
            pltpu.VMEM((CHUNK,), jnp.int32),
            pltpu.VMEM((NEG * CHUNK,), jnp.int32),
            pltpu.VMEM((CHUNK,), jnp.int32),
            pltpu.VMEM((CHUNK,), jnp.int32),
            pltpu.VMEM((NEG * CHUNK,), jnp.int32),
            pltpu.VMEM((CHUNK, D2), jnp.float32),
            pltpu.VMEM((CHUNK, D2), jnp.float32),
            pltpu.VMEM((NEG, CHUNK, D2), jnp.float32),
            pltpu.VMEM((8 * CHUNK,), jnp.float32),
            pltpu.SemaphoreType.DMA,
        ],
    )
    def sc_scores(cen_hbm, pos_hbm, negt_hbm, itab_hbm, otab_hbm, out_hbm,
                  idx_c, idx_p, idx_n, pr_c, pr_p, pr_n,
                  cen_v, pos_v, neg_v, sc_v, sem):
        wid = lax.axis_index("s") * NC + lax.axis_index("c")

        def pack(src, dst, n):
            def pb(i, carry):
                v = src[pl.ds(i * _L, _L)]
                hi = lax.shift_left(lax.shift_right_logical(v, 14), 13)
                dst[pl.ds(i * _L, _L)] = hi | jnp.bitwise_and(v, _NB - 1)
                return carry
            lax.fori_loop(0, n // _L, pb, 0)

        def chunk_body(ci, carry):
            base = wid * b_per_w + ci * CHUNK
            pltpu.sync_copy(cen_hbm.at[pl.ds(base, CHUNK)], idx_c)
            pltpu.sync_copy(pos_hbm.at[pl.ds(base, CHUNK)], idx_p)
            for k in range(NEG):
                pltpu.sync_copy(negt_hbm.at[pl.ds(k * B + base, CHUNK)],
                                idx_n.at[pl.ds(k * CHUNK, CHUNK)])
            pack(idx_c, pr_c, CHUNK)
            pack(idx_p, pr_p, CHUNK)
            pack(idx_n, pr_n, NEG * CHUNK)
            h_c = pltpu.async_copy(itab_hbm.at[pr_c], cen_v, sem)
            h_p = pltpu.async_copy(otab_hbm.at[pr_p], pos_v, sem)
            h_n = [pltpu.async_copy(
                       otab_hbm.at[pr_n.at[pl.ds(k * CHUNK, CHUNK)]],
                       neg_v.at[k], sem)
                   for k in range(NEG)]
            h_c.wait()
            h_p.wait()
            for h in h_n:
                h.wait()

            def group_body(g, carry2):
                rows = g * _L + lax.broadcasted_iota(jnp.int32, (_L,), 0)
                def off(v):
                    return lax.shift_left(
                        jnp.bitwise_and(lax.shift_right_logical(v, 13), 1), 6)
                off_c = off(idx_c[pl.ds(g * _L, _L)])
                off_p = off(idx_p[pl.ds(g * _L, _L)])
                off_n = [off(idx_n[pl.ds(k * CHUNK + g * _L, _L)])
                         for k in range(NEG)]

                def d_body(dd, accs):
                    acc_p, acc_n = accs
                    dsplat = jnp.full((_L,), dd, jnp.int32)
                    c = plsc.load_gather(cen_v, [rows, off_c + dsplat])
                    p = plsc.load_gather(pos_v, [rows, off_p + dsplat])
                    acc_p = acc_p + c * p
                    new_n = tuple(
                        acc_n[k] + c * plsc.load_gather(
                            neg_v,
                            [jnp.full((_L,), k, jnp.int32), rows,
                             off_n[k] + dsplat])
                        for k in range(NEG))
                    return acc_p, new_n

                z = jnp.zeros((_L,), jnp.float32)
                acc_p, acc_n = lax.fori_loop(0, D, d_body, (z, (z,) * NEG))
                sc_v[pl.ds(0 * CHUNK + g * _L, _L)] = acc_p
                for k in range(NEG):
                    sc_v[pl.ds((1 + k) * CHUNK + g * _L, _L)] = acc_n[k]
                sc_v[pl.ds(6 * CHUNK + g * _L, _L)] = z
                sc_v[pl.ds(7 * CHUNK + g * _L, _L)] = z
                return carry2

            lax.fori_loop(0, CHUNK // _L, group_body, 0)
            for r in range(8):
                pltpu.sync_copy(sc_v.at[pl.ds(r * CHUNK, CHUNK)],
                                out_hbm.at[pl.ds(r * B + base, CHUNK)])
            return carry

        lax.fori_loop(0, n_chunks, chunk_body, 0)

    return sc_scores


_NB = 8192


def _relayout_body(a0_ref, a1_ref, b0_ref, b1_ref, oa_ref, ob_ref):
    oa_ref[...] = jnp.concatenate(
        [jnp.transpose(a0_ref[...], (1, 0)),
         jnp.transpose(a1_ref[...], (1, 0))], axis=1)
    ob_ref[...] = jnp.concatenate(
        [jnp.transpose(b0_ref[...], (1, 0)),
         jnp.transpose(b1_ref[...], (1, 0))], axis=1)


def _relayout_pair(itab, otab, V, D):
    itab_t = jnp.transpose(itab)
    otab_t = jnp.transpose(otab)
    grid = (V + 2 * _NB - 1) // (2 * _NB)
    nblk = (V + _NB - 1) // _NB
    spec0 = pl.BlockSpec((D, _NB), lambda g: (0, 2 * g))
    spec1 = pl.BlockSpec(
        (D, _NB), lambda g: (0, jnp.minimum(2 * g + 1, nblk - 1)))
    ospec = pl.BlockSpec((_NB, 2 * D), lambda g: (g, 0))
    oshape = jax.ShapeDtypeStruct((grid * _NB, 2 * D), jnp.float32)
    return pl.pallas_call(
        _relayout_body,
        grid=(grid,),
        in_specs=[spec0, spec1, spec0, spec1],
        out_specs=[ospec, ospec],
        out_shape=[oshape, oshape],
    )(itab_t, itab_t, otab_t, otab_t)


def _loss_body(s_ref, o_ref):
    s = s_ref[...]
    rows = lax.broadcasted_iota(jnp.int32, s.shape, 0)
    x = jnp.where(rows == 0, s, -s)
    l = jnp.log(jax.nn.sigmoid(x) + 1e-10)
    l = jnp.where(rows < 6, l, 0.0)
    o_ref[0, 0] = -jnp.sum(l) / s.shape[1]


def kernel(center_words, pos_words, neg_words, input_table, output_table):
    B, = center_words.shape
    NEG = neg_words.shape[1]
    V, D = input_table.shape
    neg_t = jnp.reshape(jnp.transpose(neg_words), (-1,))
    itab2, otab2 = _relayout_pair(input_table, output_table, V, D)
    sc_scores = _make_sc_scores(B, NEG, V, D)
    scores = jnp.reshape(
        sc_scores(center_words, pos_words, neg_t, itab2, otab2), (8, B))
    loss = pl.pallas_call(
        _loss_body,
        out_shape=jax.ShapeDtypeStruct((1, 1), jnp.float32),
        out_specs=pl.BlockSpec(memory_space=pltpu.SMEM),
    )(scores)
    return jnp.reshape(loss, ())


# speedup vs baseline: 2.1497x; 1.0725x over previous
import functools

import jax
import jax.numpy as jnp
from jax import lax
from jax.experimental import pallas as pl
from jax.experimental.pallas import tpu as pltpu
from jax.experimental.pallas import tpu_sc as plsc

_L = 16


def _make_sc_scores(B, NEG, V, D):
    info = plsc.get_sparse_core_info()
    NC, NS = info.num_cores, info.num_subcores
    NW = NC * NS
    b_per_w = B // NW
    CHUNK = 64
    n_chunks = b_per_w // CHUNK
    D2 = 2 * D
    mesh = plsc.VectorSubcoreMesh(core_axis_name="c", subcore_axis_name="s")

    @functools.partial(
        pl.kernel,
        out_type=jax.ShapeDtypeStruct((8 * B,), jnp.float32),
        mesh=mesh,
        compiler_params=pltpu.CompilerParams(
            use_tc_tiling_on_sc=True, needs_layout_passes=False),
        scratch_types=[
            pltpu.VMEM((b_per_w,), jnp.int32),
            pltpu.VMEM((b_per_w,), jnp.int32),
            pltpu.VMEM((NEG * b_per_w,), jnp.int32),
            pltpu.VMEM((b_per_w,), jnp.int32),
            pltpu.VMEM((b_per_w,), jnp.int32),
            pltpu.VMEM((NEG * b_per_w,), jnp.int32),
            pltpu.VMEM((2, CHUNK, D2), jnp.float32),
            pltpu.VMEM((2, CHUNK, D2), jnp.float32),
            pltpu.VMEM((2, NEG, CHUNK, D2), jnp.float32),
            pltpu.VMEM((8 * b_per_w,), jnp.float32),
            pltpu.SemaphoreType.DMA,
            pltpu.SemaphoreType.DMA,
        ],
    )
    def sc_scores(cen_hbm, pos_hbm, negt_hbm, itab_hbm, otab_hbm, out_hbm,
                  idx_c, idx_p, idx_n, pr_c, pr_p, pr_n,
                  cen_v, pos_v, neg_v, sc_v, sem0, sem1):
        wid = lax.axis_index("s") * NC + lax.axis_index("c")
        base0 = wid * b_per_w
        sems = [sem0, sem1]

        pltpu.sync_copy(cen_hbm.at[pl.ds(base0, b_per_w)], idx_c)
        pltpu.sync_copy(pos_hbm.at[pl.ds(base0, b_per_w)], idx_p)
        for k in range(NEG):
            pltpu.sync_copy(negt_hbm.at[pl.ds(k * B + base0, b_per_w)],
                            idx_n.at[pl.ds(k * b_per_w, b_per_w)])

        def pack(src, dst, n):
            def pb(i, carry):
                v = src[pl.ds(i * _L, _L)]
                hi = lax.shift_left(lax.shift_right_logical(v, 14), 13)
                dst[pl.ds(i * _L, _L)] = hi | jnp.bitwise_and(v, _NB - 1)
                return carry
            lax.fori_loop(0, n // _L, pb, 0)

        pack(idx_c, pr_c, b_per_w)
        pack(idx_p, pr_p, b_per_w)
        pack(idx_n, pr_n, NEG * b_per_w)

        def zero_pad_rows(i, carry):
            z = jnp.zeros((_L,), jnp.float32)
            sc_v[pl.ds(6 * b_per_w + i * _L, _L)] = z
            sc_v[pl.ds(7 * b_per_w + i * _L, _L)] = z
            return carry
        lax.fori_loop(0, b_per_w // _L, zero_pad_rows, 0)

        def fire(ci):
            p = ci % 2
            hs = [pltpu.async_copy(
                      itab_hbm.at[pr_c.at[pl.ds(ci * CHUNK, CHUNK)]],
                      cen_v.at[p], sems[p]),
                  pltpu.async_copy(
                      otab_hbm.at[pr_p.at[pl.ds(ci * CHUNK, CHUNK)]],
                      pos_v.at[p], sems[p])]
            for k in range(NEG):
                hs.append(pltpu.async_copy(
                    otab_hbm.at[pr_n.at[pl.ds(k * b_per_w + ci * CHUNK,
                                              CHUNK)]],
                    neg_v.at[p, k], sems[p]))
            return hs

        def compute(ci):
            p = ci % 2

            def group_body(g, carry2):
                rows = g * _L + lax.broadcasted_iota(jnp.int32, (_L,), 0)
                def off(v):
                    return lax.shift_left(
                        jnp.bitwise_and(lax.shift_right_logical(v, 13), 1), 6)
                off_c = off(idx_c[pl.ds(ci * CHUNK + g * _L, _L)])
                off_p = off(idx_p[pl.ds(ci * CHUNK + g * _L, _L)])
                off_n = [off(idx_n[pl.ds(k * b_per_w + ci * CHUNK + g * _L,
                                         _L)])
                         for k in range(NEG)]

                def d_body(dd, accs):
                    acc_p, acc_n = accs
                    dsplat = jnp.full((_L,), dd, jnp.int32)
                    c = plsc.load_gather(cen_v.at[p], [rows, off_c + dsplat])
                    pv = plsc.load_gather(pos_v.at[p], [rows, off_p + dsplat])
                    acc_p = acc_p + c * pv
                    new_n = tuple(
                        acc_n[k] + c * plsc.load_gather(
                            neg_v.at[p],
                            [jnp.full((_L,), k, jnp.int32), rows,
                             off_n[k] + dsplat])
                        for k in range(NEG))
                    return acc_p, new_n

                z = jnp.zeros((_L,), jnp.float32)
                acc_p, acc_n = lax.fori_loop(0, D, d_body, (z, (z,) * NEG))
                sc_v[pl.ds(0 * b_per_w + ci * CHUNK + g * _L, _L)] = acc_p
                for k in range(NEG):
                    sc_v[pl.ds((1 + k) * b_per_w + ci * CHUNK + g * _L,
                               _L)] = acc_n[k]
                return carry2

            lax.fori_loop(0, CHUNK // _L, group_body, 0)

        handles = {0: fire(0)}
        for ci in range(n_chunks):
            if ci + 1 < n_chunks:
                handles[ci + 1] = fire(ci + 1)
            for h in handles.pop(ci):
                h.wait()
            compute(ci)

        for r in range(8):
            pltpu.sync_copy(sc_v.at[pl.ds(r * b_per_w, b_per_w)],
                            out_hbm.at[pl.ds(r * B + base0, b_per_w)])

    return sc_scores


_NB = 8192


def _relayout_body(a0_ref, a1_ref, b0_ref, b1_ref, oa_ref, ob_ref):
    oa_ref[...] = jnp.concatenate(
        [jnp.transpose(a0_ref[...], (1, 0)),
         jnp.transpose(a1_ref[...], (1, 0))], axis=1)
    ob_ref[...] = jnp.concatenate(
        [jnp.transpose(b0_ref[...], (1, 0)),
         jnp.transpose(b1_ref[...], (1, 0))], axis=1)


def _relayout_pair(itab, otab, V, D):
    itab_t = jnp.transpose(itab)
    otab_t = jnp.transpose(otab)
    grid = (V + 2 * _NB - 1) // (2 * _NB)
    nblk = (V + _NB - 1) // _NB
    spec0 = pl.BlockSpec((D, _NB), lambda g: (0, 2 * g))
    spec1 = pl.BlockSpec(
        (D, _NB), lambda g: (0, jnp.minimum(2 * g + 1, nblk - 1)))
    ospec = pl.BlockSpec((_NB, 2 * D), lambda g: (g, 0))
    oshape = jax.ShapeDtypeStruct((grid * _NB, 2 * D), jnp.float32)
    return pl.pallas_call(
        _relayout_body,
        grid=(grid,),
        in_specs=[spec0, spec1, spec0, spec1],
        out_specs=[ospec, ospec],
        out_shape=[oshape, oshape],
    )(itab_t, itab_t, otab_t, otab_t)


def _loss_body(s_ref, o_ref):
    s = s_ref[...]
    rows = lax.broadcasted_iota(jnp.int32, s.shape, 0)
    x = jnp.where(rows == 0, s, -s)
    l = jnp.log(jax.nn.sigmoid(x) + 1e-10)
    l = jnp.where(rows < 6, l, 0.0)
    o_ref[0, 0] = -jnp.sum(l) / s.shape[1]


def kernel(center_words, pos_words, neg_words, input_table, output_table):
    B, = center_words.shape
    NEG = neg_words.shape[1]
    V, D = input_table.shape
    neg_t = jnp.reshape(jnp.transpose(neg_words), (-1,))
    itab2, otab2 = _relayout_pair(input_table, output_table, V, D)
    sc_scores = _make_sc_scores(B, NEG, V, D)
    scores = jnp.reshape(
        sc_scores(center_words, pos_words, neg_t, itab2, otab2), (8, B))
    loss = pl.pallas_call(
        _loss_body,
        out_shape=jax.ShapeDtypeStruct((1, 1), jnp.float32),
        out_specs=pl.BlockSpec(memory_space=pltpu.SMEM),
    )(scores)
    return jnp.reshape(loss, ())
